# baseline (speedup 1.0000x reference)
import functools

import jax
import jax.numpy as jnp
from jax import lax
from jax.experimental import pallas as pl
from jax.experimental.pallas import tpu as pltpu
from jax.experimental.pallas import tpu_sc as plsc


def _att_idx_body(wk_ref, qt_ref, ut_ref, idx_ref):
    att = lax.dot_general(
        wk_ref[...], qt_ref[...],
        dimension_numbers=(((1,), (0,)), ((), ())),
        preferred_element_type=jnp.float32,
    )
    g = -jnp.log(-jnp.log(ut_ref[...]))
    z = att + g
    m = jnp.max(z, axis=0, keepdims=True)
    buf, b = z.shape
    ii = lax.broadcasted_iota(jnp.int32, (buf, b), 0)
    big = jnp.int32(2**30)
    idx = jnp.min(jnp.where(z >= m, ii, big), axis=0, keepdims=True)
    bufd = idx_ref.shape[0]
    jj = lax.broadcasted_iota(jnp.int32, (bufd, b), 0)
    idx_ref[...] = jj * jnp.int32(buf) + idx


def _adain_body(x_ref, dm_ref, ds_ref, o_ref):
    xb = x_ref[...]
    hw = xb.shape[1]
    m = jnp.sum(xb, axis=1, keepdims=True) * (1.0 / hw)
    d = xb - m
    v = jnp.sum(d * d, axis=1, keepdims=True) * (1.0 / hw)
    s = ds_ref[...] * lax.rsqrt(v + 1e-5)
    o_ref[...] = d * s + dm_ref[...]


def _make_sc_gather(n_elems, per_w, nc, ns):
    mesh = plsc.VectorSubcoreMesh(core_axis_name="c", subcore_axis_name="s")

    @functools.partial(
        pl.kernel,
        mesh=mesh,
        out_type=jax.ShapeDtypeStruct((n_elems,), jnp.float32),
        scratch_types=[
            pltpu.VMEM((per_w,), jnp.int32),
            pltpu.VMEM((per_w,), jnp.float32),
            pltpu.SemaphoreType.DMA,
        ],
    )
    def gather_k(wv_hbm, idx_hbm, out_hbm, idx_v, vals_v, sem):
        wid = lax.axis_index("s") * nc + lax.axis_index("c")
        base = wid * per_w
        pltpu.sync_copy(idx_hbm.at[pl.ds(base, per_w)], idx_v)
        pltpu.async_copy(wv_hbm.at[idx_v], vals_v, sem).wait()
        pltpu.sync_copy(vals_v, out_hbm.at[pl.ds(base, per_w)])

    return gather_k


def kernel(x, q, mean, std, Wk, Wv):
    b, c, h, w = x.shape
    hw = h * w
    buf, feat = Wk.shape
    bufd = Wv.shape[0]

    m_start = jnp.asarray(mean[1], jnp.int32) - c
    s_start = jnp.asarray(std[1], jnp.int32) - c
    m_start = m_start + (jnp.asarray(mean[0], jnp.int32) - m_start)
    s_start = s_start + (jnp.asarray(std[0], jnp.int32) - s_start)

    u = jax.random.uniform(jax.random.key(42), (b, buf), minval=1e-10, maxval=1.0)

    idx_j = pl.pallas_call(
        _att_idx_body,
        out_shape=jax.ShapeDtypeStruct((bufd, b), jnp.int32),
    )(Wk, q.T, u.T)

    n_elems = bufd * b
    info = plsc.get_sparse_core_info()
    nc, ns = info.num_cores, info.num_subcores
    per_w = n_elems // (nc * ns)
    vals = _make_sc_gather(n_elems, per_w, nc, ns)(
        Wv.reshape(-1), idx_j.reshape(-1)
    )
    di = vals.reshape(bufd, b).T

    di_mean = lax.dynamic_slice_in_dim(di, m_start, c, axis=1).reshape(b * c, 1)
    di_std = lax.dynamic_slice_in_dim(di, s_start, c, axis=1).reshape(b * c, 1)

    rb = 16
    x2 = x.reshape(b * c, hw)
    out2 = pl.pallas_call(
        _adain_body,
        grid=(b * c // rb,),
        in_specs=[
            pl.BlockSpec((rb, hw), lambda i: (i, 0)),
            pl.BlockSpec((rb, 1), lambda i: (i, 0)),
            pl.BlockSpec((rb, 1), lambda i: (i, 0)),
        ],
        out_specs=pl.BlockSpec((rb, hw), lambda i: (i, 0)),
        out_shape=jax.ShapeDtypeStruct((b * c, hw), jnp.float32),
    )(x2, di_mean, di_std)
    return out2.reshape(b, c, h, w)


# speedup vs baseline: 3.1112x; 3.1112x over previous
import functools

import jax
import jax.numpy as jnp
from jax import lax
from jax.experimental import pallas as pl
from jax.experimental.pallas import tpu as pltpu
from jax.experimental.pallas import tpu_sc as plsc


def _att_idx_body(q_ref, wk_ref, u_ref, idx_ref):
    att = lax.dot_general(
        q_ref[...], wk_ref[...],
        dimension_numbers=(((1,), (1,)), ((), ())),
        preferred_element_type=jnp.float32,
    )
    g = -jnp.log(-jnp.log(u_ref[...]))
    z = att + g
    m = jnp.max(z, axis=1, keepdims=True)
    b, buf = z.shape
    ii = lax.broadcasted_iota(jnp.int32, (b, buf), 1)
    big = jnp.int32(2**30)
    idx = jnp.min(jnp.where(z >= m, ii, big), axis=1, keepdims=True)
    bufd = idx_ref.shape[1]
    jj = lax.broadcasted_iota(jnp.int32, (b, bufd), 1)
    idx_ref[...] = jj * jnp.int32(buf) + idx


def _adain_body(x_ref, dm_ref, ds_ref, o_ref):
    xb = x_ref[...]
    hw = xb.shape[2] * xb.shape[3]
    m = jnp.sum(xb, axis=(2, 3), keepdims=True) * (1.0 / hw)
    d = xb - m
    v = jnp.sum(d * d, axis=(2, 3), keepdims=True) * (1.0 / hw)
    s = ds_ref[...] * lax.rsqrt(v + 1e-5)
    o_ref[...] = d * s + dm_ref[...]


def _make_sc_gather(n_elems, per_w, nc, ns):
    mesh = plsc.VectorSubcoreMesh(core_axis_name="c", subcore_axis_name="s")

    @functools.partial(
        pl.kernel,
        mesh=mesh,
        out_type=jax.ShapeDtypeStruct((n_elems,), jnp.float32),
        scratch_types=[
            pltpu.VMEM((per_w,), jnp.int32),
            pltpu.VMEM((per_w,), jnp.float32),
            pltpu.SemaphoreType.DMA,
        ],
    )
    def gather_k(wv_hbm, idx_hbm, out_hbm, idx_v, vals_v, sem):
        wid = lax.axis_index("s") * nc + lax.axis_index("c")
        base = wid * per_w
        pltpu.sync_copy(idx_hbm.at[pl.ds(base, per_w)], idx_v)
        pltpu.async_copy(wv_hbm.at[idx_v], vals_v, sem).wait()
        pltpu.sync_copy(vals_v, out_hbm.at[pl.ds(base, per_w)])

    return gather_k


def kernel(x, q, mean, std, Wk, Wv):
    b, c, h, w = x.shape
    buf, feat = Wk.shape
    bufd = Wv.shape[0]

    m_start = jnp.asarray(mean[1], jnp.int32) - c
    s_start = jnp.asarray(std[1], jnp.int32) - c
    m_start = m_start + (jnp.asarray(mean[0], jnp.int32) - m_start)
    s_start = s_start + (jnp.asarray(std[0], jnp.int32) - s_start)

    u = jax.random.uniform(jax.random.key(42), (b, buf), minval=1e-10, maxval=1.0)

    idx_j = pl.pallas_call(
        _att_idx_body,
        out_shape=jax.ShapeDtypeStruct((b, bufd), jnp.int32),
    )(q, Wk, u)

    n_elems = bufd * b
    info = plsc.get_sparse_core_info()
    nc, ns = info.num_cores, info.num_subcores
    per_w = n_elems // (nc * ns)
    vals = _make_sc_gather(n_elems, per_w, nc, ns)(
        Wv.reshape(-1), idx_j.reshape(-1)
    )
    di = vals.reshape(b, bufd)

    di_mean = lax.dynamic_slice_in_dim(di, m_start, c, axis=1).reshape(b, c, 1, 1)
    di_std = lax.dynamic_slice_in_dim(di, s_start, c, axis=1).reshape(b, c, 1, 1)

    ct = 16
    out = pl.pallas_call(
        _adain_body,
        grid=(b, c // ct),
        in_specs=[
            pl.BlockSpec((1, ct, h, w), lambda i, j: (i, j, 0, 0)),
            pl.BlockSpec((1, ct, 1, 1), lambda i, j: (i, j, 0, 0)),
            pl.BlockSpec((1, ct, 1, 1), lambda i, j: (i, j, 0, 0)),
        ],
        out_specs=pl.BlockSpec((1, ct, h, w), lambda i, j: (i, j, 0, 0)),
        out_shape=jax.ShapeDtypeStruct((b, c, h, w), jnp.float32),
    )(x, di_mean, di_std)
    return out
